# baseline (speedup 1.0000x reference)
import functools

import jax
import jax.numpy as jnp
from jax import lax
from jax.experimental import pallas as pl
from jax.experimental.pallas import tpu as pltpu
from jax.experimental.pallas import tpu_sc as plsc

_VOCAB = 1000000
_EMBED = 64
_B = 4096
_L = 200
_N = _B * _L
_LP = _L // 2

_GATHER_WINDOW = 128


def _sc_gather(token_table, flat_ids):
    mesh = plsc.VectorSubcoreMesh(core_axis_name="c", subcore_axis_name="s")

    @functools.partial(
        pl.kernel,
        out_type=jax.ShapeDtypeStruct((_N, _EMBED), jnp.float32),
        mesh=mesh,
        compiler_params=pltpu.CompilerParams(use_tc_tiling_on_sc=False),
    )
    def gather_kernel(table_hbm, idx_hbm, out_hbm):
        def body(i_vmem, o_vmem):
            pltpu.sync_copy(table_hbm.at[i_vmem.at[0]], o_vmem)

        pltpu.emit_pipeline(
            body,
            grid=(_N // _GATHER_WINDOW,),
            in_specs=[
                pl.BlockSpec((1, _GATHER_WINDOW), index_map=lambda i: (0, i))
            ],
            out_specs=[
                pl.BlockSpec((_GATHER_WINDOW, _EMBED), index_map=lambda i: (i, 0))
            ],
            core_axis_name=("c", "s"),
            dimension_semantics=(pltpu.PARALLEL,),
        )(idx_hbm, out_hbm)

    return gather_kernel(token_table, flat_ids.reshape(1, _N))


_BP = 4
_BC = 512


def _ln_p_body(tok_ref, pos_ref, gamma_ref, beta_ref, out_ref):
    y = tok_ref[...] + pos_ref[...]
    g = gamma_ref[...]
    b = beta_ref[...]
    for h in (slice(0, _EMBED), slice(_EMBED, 128)):
        yh = y[:, h, :]
        m = jnp.mean(yh, axis=1, keepdims=True)
        q = jnp.mean(yh * yh, axis=1, keepdims=True)
        r = lax.rsqrt(q - m * m + 1e-5)
        out_ref[:, h, :] = (yh - m) * r * g[:, h, :] + b[:, h, :]


def _tc_layernorm_p(tokP, posP, gP, bP):
    return pl.pallas_call(
        _ln_p_body,
        grid=(_LP // _BP, _B // _BC),
        in_specs=[
            pl.BlockSpec((_BP, 128, _BC), lambda i, j: (i, 0, j)),
            pl.BlockSpec((_BP, 128, 1), lambda i, j: (i, 0, 0)),
            pl.BlockSpec((1, 128, 1), lambda i, j: (0, 0, 0)),
            pl.BlockSpec((1, 128, 1), lambda i, j: (0, 0, 0)),
        ],
        out_specs=pl.BlockSpec((_BP, 128, _BC), lambda i, j: (i, 0, j)),
        out_shape=jax.ShapeDtypeStruct((_LP, 128, _B), jnp.float32),
    )(tokP, posP, gP, bP)


def kernel(input_ids, token_table, pos_table, gamma, beta):
    flat_ids = input_ids.reshape(-1).astype(jnp.int32)
    tok2 = _sc_gather(token_table, flat_ids)
    tokM = lax.optimization_barrier(tok2.reshape(_N // 2, 128))
    tokP = jnp.transpose(tokM.reshape(_B, _LP, 128), (1, 2, 0))
    posP = pos_table[:_L].reshape(_LP, 128, 1)
    gP = jnp.tile(gamma, 2).reshape(1, 128, 1)
    bP = jnp.tile(beta, 2).reshape(1, 128, 1)
    outP = _tc_layernorm_p(tokP, posP, gP, bP)
    return jnp.transpose(outP.reshape(_L, _EMBED, _B), (2, 0, 1))
